# Initial kernel scaffold; baseline (speedup 1.0000x reference)
import jax
import jax.numpy as jnp
from jax.experimental import pallas as pl


def kernel(node_type, velocity, cells, mesh_pos, params, is_trainning):
    raise NotImplementedError("write your pallas kernel here")


# speedup vs baseline: 1.2239x; 1.2239x over previous
import functools

import jax
import jax.numpy as jnp
from jax import lax
from jax.experimental import pallas as pl
from jax.experimental.pallas import tpu as pltpu
from jax.experimental.pallas import tpu_sc as plsc

N_NODES = 10000
N_PAD = 10240
NODE_TYPE_SIZE = 9
LATENT = 256
STEPS = 15
OUT = 2

E = 60000
E_PAD = 61440
G = 2 * E_PAD

_NC = 2
_NS = 16
_NW = _NC * _NS
_CH = 128

_HALF = N_NODES // _NC
_SP_ROWS = 5056
_JUNK_ROW = 5000

_BLK = 512


def _sc_gather(table, idx, d):
    b = idx.shape[0]
    per_w = b // _NW
    chg = 64
    n_ch = per_w // chg
    nb = 6
    mesh = plsc.VectorSubcoreMesh(core_axis_name="c", subcore_axis_name="s")

    def body(table_hbm, idx_hbm, out_hbm, idx_all, *bufs_and_sems):
        bufs = bufs_and_sems[:nb]
        gsems = bufs_and_sems[nb:2 * nb]
        wsems = bufs_and_sems[2 * nb:3 * nb]
        wid = lax.axis_index("s") * _NC + lax.axis_index("c")
        base = wid * per_w
        pltpu.sync_copy(idx_hbm.at[pl.ds(base, per_w)], idx_all)

        for k in range(nb):
            pltpu.async_copy(table_hbm.at[idx_all.at[pl.ds(k * chg, chg)]],
                             bufs[k], gsems[k])

        def grp(g, carry):
            for k in range(nb):
                i = g * nb + k
                pltpu.make_async_copy(
                    table_hbm.at[idx_all.at[pl.ds(0, chg)]], bufs[k],
                    gsems[k]).wait()
                pltpu.async_copy(bufs[k],
                                 out_hbm.at[pl.ds(base + i * chg, chg)],
                                 wsems[k])

                @pl.when(i + nb < n_ch)
                def _():
                    pltpu.make_async_copy(
                        bufs[k], out_hbm.at[pl.ds(base, chg)],
                        wsems[k]).wait()
                    pltpu.async_copy(
                        table_hbm.at[idx_all.at[pl.ds((i + nb) * chg, chg)]],
                        bufs[k], gsems[k])

            return carry

        lax.fori_loop(0, n_ch // nb, grp, 0)
        for k in range(nb):
            pltpu.make_async_copy(bufs[k], out_hbm.at[pl.ds(base, chg)],
                                  wsems[k]).wait()

    run = pl.kernel(
        body,
        out_type=jax.ShapeDtypeStruct((b, d), jnp.float32),
        mesh=mesh,
        scratch_types=(
            [pltpu.VMEM((per_w,), jnp.int32)]
            + [pltpu.VMEM((chg, d), jnp.float32)] * nb
            + [pltpu.SemaphoreType.DMA] * (2 * nb)
        ),
    )
    return run(table, idx)


_FW = LATENT // _NW


def _sc_scatter_add(ne_t, ridx):
    ch = 512
    n_ch = E_PAD // ch
    nb = 2
    mesh = plsc.VectorSubcoreMesh(core_axis_name="c", subcore_axis_name="s")

    def body(net_hbm, ridx_hbm, out_hbm, i0, i1, d0, d1, acc,
             si0, si1, sd0, sd1, so):
        wid = lax.axis_index("s") * _NC + lax.axis_index("c")
        f0 = wid * _FW
        ibufs = (i0, i1)
        dbufs = (d0, d1)
        isems = (si0, si1)
        dsems = (sd0, sd1)

        def start(i, k):
            pltpu.async_copy(ridx_hbm.at[pl.ds(i * ch, ch)], ibufs[k],
                             isems[k])
            pltpu.async_copy(net_hbm.at[pl.ds(f0, _FW), pl.ds(i * ch, ch)],
                             dbufs[k], dsems[k])

        for k in range(nb):
            start(k, k)

        def z16(i, carry):
            base = i * 128
            for u in range(8):
                acc[pl.ds(base + u * 16, 16)] = jnp.zeros((16,), jnp.float32)
            return carry

        lax.fori_loop(0, (N_PAD * _FW) // 128, z16, 0)

        def grp(g, carry):
            for k in range(nb):
                i = g * nb + k
                pltpu.make_async_copy(ridx_hbm.at[pl.ds(0, ch)], ibufs[k],
                                      isems[k]).wait()
                pltpu.make_async_copy(
                    net_hbm.at[pl.ds(f0, _FW), pl.ds(0, ch)], dbufs[k],
                    dsems[k]).wait()
                for j in range(ch // 16):
                    rowv = ibufs[k][pl.ds(j * 16, 16)] * _FW
                    for f in range(_FW):
                        plsc.addupdate_scatter(
                            acc, [rowv + f],
                            dbufs[k][f, pl.ds(j * 16, 16)])

                @pl.when(i + nb < n_ch)
                def _():
                    start(i + nb, k)

            return carry

        lax.fori_loop(0, n_ch // nb, grp, 0)
        pltpu.async_copy(acc, out_hbm.at[wid], so)
        pltpu.make_async_copy(acc, out_hbm.at[wid], so).wait()

    run = pl.kernel(
        body,
        out_type=jax.ShapeDtypeStruct((_NW, N_PAD * _FW), jnp.float32),
        mesh=mesh,
        compiler_params=pltpu.CompilerParams(needs_layout_passes=False),
        scratch_types=[
            pltpu.VMEM((ch,), jnp.int32),
            pltpu.VMEM((ch,), jnp.int32),
            pltpu.VMEM((_FW, ch), jnp.float32),
            pltpu.VMEM((_FW, ch), jnp.float32),
            pltpu.VMEM((N_PAD * _FW,), jnp.float32),
            pltpu.SemaphoreType.DMA,
            pltpu.SemaphoreType.DMA,
            pltpu.SemaphoreType.DMA,
            pltpu.SemaphoreType.DMA,
            pltpu.SemaphoreType.DMA,
        ],
    )
    return run(ne_t, ridx)


def _dot(a, b):
    return jnp.dot(a, b, preferred_element_type=jnp.float32)


def _mlp_tail(h1, w2, b2, w3, b3, ln):
    h1 = jnp.maximum(h1, 0.0)
    h2 = jnp.maximum(_dot(h1, w2) + b2, 0.0)
    h3 = _dot(h2, w3) + b3
    if ln:
        mu = jnp.mean(h3, axis=-1, keepdims=True)
        xc = h3 - mu
        var = jnp.mean(xc * xc, axis=-1, keepdims=True)
        h3 = xc * lax.rsqrt(var + 1e-5)
    return h3


def _full(shape):
    return pl.BlockSpec(shape, lambda i: (0, 0))


def _rows(blk, w, off=0):
    return pl.BlockSpec((blk, w), lambda i, off=off: (i + off, 0))


def _edge_step(el, gath, p):
    (w1, b1), (w2, b2), (w3, b3) = p
    w1e, w1s, w1r = w1[:LATENT], w1[LATENT:2 * LATENT], w1[2 * LATENT:]
    grid = E_PAD // _BLK

    def body(el_ref, sf_ref, rf_ref, w1e_ref, w1s_ref, w1r_ref, b1_ref,
             w2_ref, b2_ref, w3_ref, b3_ref, net_ref, elo_ref):
        el_b = el_ref[...]
        h1 = (_dot(el_b, w1e_ref[...]) + _dot(sf_ref[...], w1s_ref[...])
              + _dot(rf_ref[...], w1r_ref[...]) + b1_ref[...])
        h3 = _mlp_tail(h1, w2_ref[...], b2_ref[...], w3_ref[...], b3_ref[...],
                       True)
        rows = (pl.program_id(0) * _BLK
                + lax.broadcasted_iota(jnp.int32, (_BLK, 1), 0))
        ne = h3 * (rows < E).astype(jnp.float32)
        net_ref[...] = ne.T
        elo_ref[...] = el_b + ne

    return pl.pallas_call(
        body,
        grid=(grid,),
        in_specs=[
            _rows(_BLK, LATENT),
            _rows(_BLK, LATENT),
            _rows(_BLK, LATENT, off=grid),
            _full((LATENT, LATENT)), _full((LATENT, LATENT)),
            _full((LATENT, LATENT)), _full((1, LATENT)),
            _full((LATENT, LATENT)), _full((1, LATENT)),
            _full((LATENT, LATENT)), _full((1, LATENT)),
        ],
        out_specs=[pl.BlockSpec((LATENT, _BLK), lambda i: (0, i)),
                   _rows(_BLK, LATENT)],
        out_shape=[
            jax.ShapeDtypeStruct((LATENT, E_PAD), jnp.float32),
            jax.ShapeDtypeStruct((E_PAD, LATENT), jnp.float32),
        ],
    )(el, gath, gath, w1e, w1s, w1r, b1.reshape(1, -1), w2,
      b2.reshape(1, -1), w3, b3.reshape(1, -1))


def _node_step(nl, agg, p):
    (w1, b1), (w2, b2), (w3, b3) = p
    w1n, w1a = w1[:LATENT], w1[LATENT:]
    grid = N_PAD // _BLK

    def body(nl_ref, agg_ref, w1n_ref, w1a_ref, b1_ref, w2_ref, b2_ref,
             w3_ref, b3_ref, out_ref):
        nl_b = nl_ref[...]
        h1 = (_dot(nl_b, w1n_ref[...]) + _dot(agg_ref[...], w1a_ref[...])
              + b1_ref[...])
        h3 = _mlp_tail(h1, w2_ref[...], b2_ref[...], w3_ref[...], b3_ref[...],
                       True)
        out_ref[...] = nl_b + h3

    return pl.pallas_call(
        body,
        grid=(grid,),
        in_specs=[
            _rows(_BLK, LATENT),
            _rows(_BLK, LATENT),
            _full((LATENT, LATENT)), _full((LATENT, LATENT)),
            _full((1, LATENT)),
            _full((LATENT, LATENT)), _full((1, LATENT)),
            _full((LATENT, LATENT)), _full((1, LATENT)),
        ],
        out_specs=_rows(_BLK, LATENT),
        out_shape=jax.ShapeDtypeStruct((N_PAD, LATENT), jnp.float32),
    )(nl, agg, w1n, w1a, b1.reshape(1, -1), w2, b2.reshape(1, -1), w3,
      b3.reshape(1, -1))


def _edge_stats(posg):
    grid = E_PAD // _BLK

    def body(sf_ref, rf_ref, out_ref):
        @pl.when(pl.program_id(0) == 0)
        def _():
            out_ref[...] = jnp.zeros_like(out_ref)

        relx = sf_ref[:, 0:1] - rf_ref[:, 0:1]
        rely = sf_ref[:, 1:2] - rf_ref[:, 1:2]
        d = jnp.sqrt(relx * relx + rely * rely)
        rows = (pl.program_id(0) * _BLK
                + lax.broadcasted_iota(jnp.int32, (_BLK, 1), 0))
        m = (rows < E).astype(jnp.float32)
        relx = relx * m
        rely = rely * m
        d = d * m
        lane = lax.broadcasted_iota(jnp.int32, (1, 128), 1)
        sv = (jnp.sum(relx) * (lane == 0) + jnp.sum(rely) * (lane == 1)
              + jnp.sum(d) * (lane == 2)).astype(jnp.float32)
        qv = (jnp.sum(relx * relx) * (lane == 0)
              + jnp.sum(rely * rely) * (lane == 1)
              + jnp.sum(d * d) * (lane == 2)).astype(jnp.float32)
        out_ref[0:1, :] += sv
        out_ref[1:2, :] += qv

    return pl.pallas_call(
        body,
        grid=(grid,),
        in_specs=[_rows(_BLK, 128), _rows(_BLK, 128, off=grid)],
        out_specs=pl.BlockSpec((8, 128), lambda i: (0, 0)),
        out_shape=jax.ShapeDtypeStruct((8, 128), jnp.float32),
    )(posg, posg)


def _node_stats(nf):
    grid = N_PAD // _BLK

    def body(nf_ref, out_ref):
        @pl.when(pl.program_id(0) == 0)
        def _():
            out_ref[...] = jnp.zeros_like(out_ref)

        rows = (pl.program_id(0) * _BLK
                + lax.broadcasted_iota(jnp.int32, (_BLK, 1), 0))
        m = (rows < N_NODES).astype(jnp.float32)
        lane = lax.broadcasted_iota(jnp.int32, (1, 128), 1)
        sv = jnp.zeros((1, 128), jnp.float32)
        qv = jnp.zeros((1, 128), jnp.float32)
        for k in range(2 + NODE_TYPE_SIZE):
            col = nf_ref[:, k:k + 1] * m
            sv = sv + jnp.sum(col) * (lane == k)
            qv = qv + jnp.sum(col * col) * (lane == k)
        out_ref[0:1, :] += sv
        out_ref[1:2, :] += qv

    return pl.pallas_call(
        body,
        grid=(grid,),
        in_specs=[_rows(_BLK, 16)],
        out_specs=pl.BlockSpec((8, 128), lambda i: (0, 0)),
        out_shape=jax.ShapeDtypeStruct((8, 128), jnp.float32),
    )(nf)


def _edge_enc(posg, stats, p):
    (w1, b1), (w2, b2), (w3, b3) = p
    w1p = jnp.pad(w1, ((0, 5), (0, 0)))
    grid = E_PAD // _BLK

    def body(sf_ref, rf_ref, st_ref, w1_ref, b1_ref, w2_ref, b2_ref, w3_ref,
             b3_ref, out_ref):
        relx = sf_ref[:, 0:1] - rf_ref[:, 0:1]
        rely = sf_ref[:, 1:2] - rf_ref[:, 1:2]
        d = jnp.sqrt(relx * relx + rely * rely)
        fx = (relx - st_ref[0, 0]) * st_ref[1, 0]
        fy = (rely - st_ref[0, 1]) * st_ref[1, 1]
        fd = (d - st_ref[0, 2]) * st_ref[1, 2]
        h1 = (fx * w1_ref[0:1, :] + fy * w1_ref[1:2, :] + fd * w1_ref[2:3, :]
              + b1_ref[...])
        out_ref[...] = _mlp_tail(h1, w2_ref[...], b2_ref[...], w3_ref[...],
                                 b3_ref[...], True)

    return pl.pallas_call(
        body,
        grid=(grid,),
        in_specs=[
            _rows(_BLK, 128), _rows(_BLK, 128, off=grid),
            _full((8, 128)),
            _full((8, LATENT)), _full((1, LATENT)),
            _full((LATENT, LATENT)), _full((1, LATENT)),
            _full((LATENT, LATENT)), _full((1, LATENT)),
        ],
        out_specs=_rows(_BLK, LATENT),
        out_shape=jax.ShapeDtypeStruct((E_PAD, LATENT), jnp.float32),
    )(posg, posg, stats, w1p, b1.reshape(1, -1), w2, b2.reshape(1, -1), w3,
      b3.reshape(1, -1))


def _node_enc(nf, stats, p):
    (w1, b1), (w2, b2), (w3, b3) = p
    nfeat = 2 + NODE_TYPE_SIZE
    w1p = jnp.pad(w1, ((0, 16 - nfeat), (0, 0)))
    grid = N_PAD // _BLK

    def body(nf_ref, st_ref, w1_ref, b1_ref, w2_ref, b2_ref, w3_ref, b3_ref,
             out_ref):
        h1 = jnp.broadcast_to(b1_ref[...], (_BLK, LATENT))
        for k in range(nfeat):
            xk = (nf_ref[:, k:k + 1] - st_ref[0, k]) * st_ref[1, k]
            h1 = h1 + xk * w1_ref[k:k + 1, :]
        out_ref[...] = _mlp_tail(h1, w2_ref[...], b2_ref[...], w3_ref[...],
                                 b3_ref[...], True)

    return pl.pallas_call(
        body,
        grid=(grid,),
        in_specs=[
            _rows(_BLK, 16),
            _full((8, 128)),
            _full((16, LATENT)), _full((1, LATENT)),
            _full((LATENT, LATENT)), _full((1, LATENT)),
            _full((LATENT, LATENT)), _full((1, LATENT)),
        ],
        out_specs=_rows(_BLK, LATENT),
        out_shape=jax.ShapeDtypeStruct((N_PAD, LATENT), jnp.float32),
    )(nf, stats, w1p, b1.reshape(1, -1), w2, b2.reshape(1, -1), w3,
      b3.reshape(1, -1))


def _decoder(nl, p):
    (w1, b1), (w2, b2), (w3, b3) = p
    w3p = jnp.pad(w3, ((0, 0), (0, 128 - OUT)))
    b3p = jnp.pad(b3, ((0, 128 - OUT),))
    grid = N_PAD // _BLK

    def body(nl_ref, w1_ref, b1_ref, w2_ref, b2_ref, w3_ref, b3_ref, out_ref):
        h1 = _dot(nl_ref[...], w1_ref[...]) + b1_ref[...]
        out_ref[...] = _mlp_tail(h1, w2_ref[...], b2_ref[...], w3_ref[...],
                                 b3_ref[...], False)

    return pl.pallas_call(
        body,
        grid=(grid,),
        in_specs=[
            _rows(_BLK, LATENT),
            _full((LATENT, LATENT)), _full((1, LATENT)),
            _full((LATENT, LATENT)), _full((1, LATENT)),
            _full((LATENT, 128)), _full((1, 128)),
        ],
        out_specs=_rows(_BLK, 128),
        out_shape=jax.ShapeDtypeStruct((N_PAD, 128), jnp.float32),
    )(nl, w1, b1.reshape(1, -1), w2, b2.reshape(1, -1), w3p,
      b3p.reshape(1, -1))


def _finalize_stats(acc, count):
    mean = acc[0, :] / count
    var = jnp.maximum(acc[1, :] / count - mean * mean, 0.0)
    rstd = 1.0 / jnp.maximum(jnp.sqrt(var), 1e-8)
    return jnp.zeros((8, 128), jnp.float32).at[0].set(mean).at[1].set(rstd)


def kernel(node_type, velocity, cells, mesh_pos, params, is_trainning=True):
    s0 = jnp.concatenate([cells[:, 0], cells[:, 1], cells[:, 2]])
    r0 = jnp.concatenate([cells[:, 1], cells[:, 2], cells[:, 0]])
    senders = jnp.concatenate([s0, r0])
    receivers = jnp.concatenate([r0, s0])
    zpad = jnp.zeros((E_PAD - E,), jnp.int32)
    s_pad = jnp.concatenate([senders, zpad])
    r_pad = jnp.concatenate([receivers, zpad])
    gidx = jnp.concatenate([s_pad, r_pad])

    nt = jax.nn.one_hot(node_type[:, 0], NODE_TYPE_SIZE, dtype=jnp.float32)
    nf = jnp.concatenate(
        [velocity, nt, jnp.zeros((N_NODES, 16 - 2 - NODE_TYPE_SIZE))], axis=1)
    nf = jnp.pad(nf, ((0, N_PAD - N_NODES), (0, 0)))

    mp128 = jnp.pad(mesh_pos, ((0, 0), (0, 126)))
    posg = _sc_gather(mp128, gidx, 128)

    e_stats = _finalize_stats(_edge_stats(posg), float(E))
    n_stats = _finalize_stats(_node_stats(nf), float(N_NODES))

    el = _edge_enc(posg, e_stats, params['edge_enc'])
    nl = _node_enc(nf, n_stats, params['node_enc'])

    for i in range(STEPS):
        gath = _sc_gather(nl, gidx, LATENT)
        ne_t, el = _edge_step(el, gath, params['edge_blocks'][i])
        acc = _sc_scatter_add(ne_t, r_pad)
        agg = (acc.reshape(_NW, N_PAD, _FW).transpose(1, 0, 2)
               .reshape(N_PAD, LATENT))
        nl = _node_step(nl, agg, params['node_blocks'][i])

    out = _decoder(nl, params['decoder'])
    return out[:N_NODES, :OUT]
